# Initial kernel scaffold; baseline (speedup 1.0000x reference)
import jax
import jax.numpy as jnp
from jax.experimental import pallas as pl


def kernel(inputs, dutyCycle):
    raise NotImplementedError("write your pallas kernel here")


# speedup vs baseline: 11.0473x; 11.0473x over previous
import functools

import jax
import jax.numpy as jnp
from jax import lax
from jax.experimental import pallas as pl
from jax.experimental.pallas import tpu as pltpu
from jax.experimental.pallas import tpu_sc as plsc

_BATCH = 128
_CH = 32768
_KWIN = 3276
_BOOST_STRENGTH = 1.0
_TARGET_DUTY = float(_KWIN) / float(_CH)

_NW = 32
_ROWS_PER_W = _BATCH // _NW
_NCHUNK = _CH // 16


def _bf_body(dc_ref, bf_ref):
    bf_ref[...] = jnp.exp((_TARGET_DUTY - dc_ref[...]) * _BOOST_STRENGTH)


def _boost_factors(dutyCycle):
    dc2 = dutyCycle.reshape(256, 128)
    bf2 = pl.pallas_call(
        _bf_body,
        out_shape=jax.ShapeDtypeStruct((256, 128), jnp.float32),
    )(dc2)
    return bf2.reshape(_CH)


def _float_key(bits):
    return jnp.where(bits >= 0, bits, bits ^ jnp.int32(0x7FFFFFFF))


def _boundary(hist_ref, nbins, k_target):
    nch = nbins // 16

    def body(c, carry):
        cum, n_ge, cnt_lt = carry
        base = nbins - 16 * (c + 1)
        chunk = hist_ref[pl.ds(base, 16)]
        rv = lax.rev(chunk, (0,))
        cs = plsc.cumsum(rv) + cum
        ge = (cs >= k_target).astype(jnp.int32)
        n_ge = n_ge + jnp.sum(ge)
        cnt_lt = cnt_lt + jnp.sum(rv * (1 - ge))
        cum = cum + jnp.sum(chunk)
        return cum, n_ge, cnt_lt

    z = jnp.int32(0)
    _, n_ge, cnt_lt = lax.fori_loop(0, nch, body, (z, z, z))
    return n_ge - 1, k_target - cnt_lt


def _sc_body(x_hbm, bf_hbm, out_hbm, rowbuf, bfbuf, keybuf, h1, h2, h3, tbuf):
    wid = lax.axis_index("s") * 2 + lax.axis_index("c")
    pltpu.sync_copy(bf_hbm, bfbuf)
    ones = jnp.ones((16,), jnp.int32)
    zeros16 = jnp.zeros((16,), jnp.int32)
    lane = lax.iota(jnp.int32, 16)

    def row_body(j, tvec):
        row = wid * _ROWS_PER_W + j
        pltpu.sync_copy(x_hbm.at[row], rowbuf)

        def zero12(i, _):
            h1[pl.ds(i * 16, 16)] = zeros16
            h2[pl.ds(i * 16, 16)] = zeros16
            return 0

        lax.fori_loop(0, 256, zero12, 0)

        def zero3(i, _):
            h3[pl.ds(i * 16, 16)] = zeros16
            return 0

        lax.fori_loop(0, 16, zero3, 0)

        def scan1(i, _):
            x = rowbuf[pl.ds(i * 16, 16)]
            b = bfbuf[pl.ds(i * 16, 16)]
            key = _float_key(lax.bitcast_convert_type(x * b, jnp.int32))
            keybuf[pl.ds(i * 16, 16)] = key
            plsc.addupdate_scatter(h1, [(key >> 20) + 2048], ones)
            return 0

        lax.fori_loop(0, _NCHUNK, scan1, 0)
        b1, r1 = _boundary(h1, 4096, _KWIN)

        def scan2(i, _):
            key = keybuf[pl.ds(i * 16, 16)]
            pred = ((key >> 20) + 2048) == b1
            plsc.addupdate_scatter(h2, [(key >> 8) & 0xFFF], ones, mask=pred)
            return 0

        lax.fori_loop(0, _NCHUNK, scan2, 0)
        b2, r2 = _boundary(h2, 4096, r1)

        p24 = ((b1 - 2048) << 12) | b2

        def scan3(i, _):
            key = keybuf[pl.ds(i * 16, 16)]
            pred = (key >> 8) == p24
            plsc.addupdate_scatter(h3, [key & 0xFF], ones, mask=pred)
            return 0

        lax.fori_loop(0, _NCHUNK, scan3, 0)
        b3, _r3 = _boundary(h3, 256, r2)

        tkey = ((b1 - 2048) << 20) | (b2 << 8) | b3
        return jnp.where(lane == j, tkey, tvec)

    tvec = lax.fori_loop(0, _ROWS_PER_W, row_body, jnp.zeros((16,), jnp.int32))
    tbuf[...] = tvec
    pltpu.sync_copy(tbuf, out_hbm.at[wid])


_sc_thresholds = functools.partial(
    pl.kernel,
    out_type=jax.ShapeDtypeStruct((_NW, 16), jnp.int32),
    mesh=plsc.VectorSubcoreMesh(core_axis_name="c", subcore_axis_name="s"),
    compiler_params=pltpu.CompilerParams(needs_layout_passes=False),
    scratch_types=[
        pltpu.VMEM((_CH,), jnp.float32),
        pltpu.VMEM((_CH,), jnp.float32),
        pltpu.VMEM((_CH,), jnp.int32),
        pltpu.VMEM((4096,), jnp.int32),
        pltpu.VMEM((4096,), jnp.int32),
        pltpu.VMEM((256,), jnp.int32),
        pltpu.VMEM((16,), jnp.int32),
    ],
)(_sc_body)


def _mask_body(x_ref, bf_ref, t_ref, o_ref):
    x = x_ref[...]
    bits = lax.bitcast_convert_type(x * bf_ref[...], jnp.int32)
    key = _float_key(bits)
    o_ref[...] = jnp.where(key >= t_ref[...], x, jnp.float32(0.0))


def _apply_mask(inputs, bf, thr):
    rb = 8
    grid = _BATCH // rb
    return pl.pallas_call(
        _mask_body,
        grid=(grid,),
        in_specs=[
            pl.BlockSpec((rb, _CH), lambda i: (i, 0)),
            pl.BlockSpec((1, _CH), lambda i: (0, 0)),
            pl.BlockSpec((rb, 1), lambda i: (i, 0)),
        ],
        out_specs=pl.BlockSpec((rb, _CH), lambda i: (i, 0)),
        out_shape=jax.ShapeDtypeStruct((_BATCH, _CH), jnp.float32),
    )(inputs, bf.reshape(1, _CH), thr)


@jax.jit
def kernel(inputs, dutyCycle):
    bf = _boost_factors(dutyCycle)
    tmat = _sc_thresholds(inputs, bf)
    thr = tmat[:, :_ROWS_PER_W].reshape(_BATCH, 1)
    return _apply_mask(inputs, bf, thr)


# speedup vs baseline: 11.1518x; 1.0095x over previous
import functools

import jax
import jax.numpy as jnp
from jax import lax
from jax.experimental import pallas as pl
from jax.experimental.pallas import tpu as pltpu
from jax.experimental.pallas import tpu_sc as plsc

_BATCH = 128
_CH = 32768
_KWIN = 3276
_BOOST_STRENGTH = 1.0
_TARGET_DUTY = float(_KWIN) / float(_CH)

_NW = 32
_ROWS_PER_W = _BATCH // _NW
_NCHUNK = _CH // 16
_UNROLL = 8


def _bf_body(dc_ref, bf_ref):
    bf_ref[...] = jnp.exp((_TARGET_DUTY - dc_ref[...]) * _BOOST_STRENGTH)


def _boost_factors(dutyCycle):
    dc2 = dutyCycle.reshape(256, 128)
    bf2 = pl.pallas_call(
        _bf_body,
        out_shape=jax.ShapeDtypeStruct((256, 128), jnp.float32),
    )(dc2)
    return bf2.reshape(_CH)


def _float_key(bits):
    return bits ^ ((bits >> 31) & jnp.int32(0x7FFFFFFF))


def _search_chunks(chunks, k_target, reverse=None):
    n = len(chunks)
    rvs = [lax.rev(ch, (0,)) for ch in chunks]
    css = [plsc.cumsum(rv) for rv in rvs]
    tots = [jnp.sum(ch) for ch in chunks]
    above = [None] * n
    acc = jnp.int32(0)
    for c in range(n - 1, -1, -1):
        above[c] = acc
        acc = acc + tots[c]
    n_ge = jnp.zeros((16,), jnp.int32)
    cnt_lt = jnp.zeros((16,), jnp.int32)
    for c in range(n):
        cs_full = css[c] + above[c]
        ge = (cs_full >= k_target).astype(jnp.int32)
        n_ge = n_ge + ge
        cnt_lt = cnt_lt + rvs[c] * (1 - ge)
    return jnp.sum(n_ge) - 1, k_target - jnp.sum(cnt_lt)


def _search_4096(fine_ref, coarse_ref, k_target):
    coarse = [coarse_ref[pl.ds(16 * c, 16)] for c in range(16)]
    g, rg = _search_chunks(coarse, k_target)
    chunk = fine_ref[pl.ds(pl.multiple_of(g * 16, 16), 16)]
    lb, r = _search_chunks([chunk], rg)
    return g * 16 + lb, r


def _search_256(h_ref, k_target):
    chunks = [h_ref[pl.ds(16 * c, 16)] for c in range(16)]
    return _search_chunks(chunks, k_target)


def _sc_body(x_hbm, bf_hbm, out_hbm, buf0, buf1, bfbuf, h1, h1c, h2, h2c,
             h3, tbuf, sem0, sem1, bsem):
    wid = lax.axis_index("s") * 2 + lax.axis_index("c")
    row0 = wid * _ROWS_PER_W
    bufs = (buf0, buf1)
    sems = (sem0, sem1)
    cp_bf = pltpu.async_copy(bf_hbm, bfbuf, bsem)
    copies = [pltpu.async_copy(x_hbm.at[row0], buf0, sem0),
              pltpu.async_copy(x_hbm.at[row0 + 1], buf1, sem1)]
    cp_bf.wait()

    ones = jnp.ones((16,), jnp.int32)
    zeros16 = jnp.zeros((16,), jnp.int32)
    zf16 = jnp.zeros((16,), jnp.float32)
    lane = lax.iota(jnp.int32, 16)
    tvec = zeros16

    for j in range(_ROWS_PER_W):
        rowbuf = bufs[j % 2]
        copies[j].wait()

        def zero12(i, _):
            for u in range(_UNROLL):
                off = i * 16 * _UNROLL + u * 16
                h1[pl.ds(off, 16)] = zeros16
                h2[pl.ds(off, 16)] = zeros16
            return 0

        lax.fori_loop(0, 256 // _UNROLL, zero12, 0)
        for c in range(16):
            h1c[pl.ds(16 * c, 16)] = zeros16
            h2c[pl.ds(16 * c, 16)] = zeros16
            h3[pl.ds(16 * c, 16)] = zeros16

        def scan1(i, _):
            for u in range(_UNROLL):
                off = i * 16 * _UNROLL + u * 16
                x = rowbuf[pl.ds(off, 16)]
                b = bfbuf[pl.ds(off, 16)]
                key = _float_key(lax.bitcast_convert_type(x * b, jnp.int32))
                rowbuf[pl.ds(off, 16)] = lax.bitcast_convert_type(
                    key, jnp.float32)
                bin1 = (key >> 20) + 2048
                plsc.addupdate_scatter(h1, [bin1], ones)
                plsc.addupdate_scatter(h1c, [bin1 >> 4], ones)
            return 0

        lax.fori_loop(0, _NCHUNK // _UNROLL, scan1, 0)
        b1, r1 = _search_4096(h1, h1c, _KWIN)
        s1 = b1 - 2048

        def scan2(i, _):
            for u in range(_UNROLL):
                off = i * 16 * _UNROLL + u * 16
                key = lax.bitcast_convert_type(
                    rowbuf[pl.ds(off, 16)], jnp.int32)
                pred = (key >> 20) == s1
                bin2 = (key >> 8) & 0xFFF
                plsc.addupdate_scatter(h2, [bin2], ones, mask=pred)
                plsc.addupdate_scatter(h2c, [bin2 >> 4], ones, mask=pred)
            return 0

        lax.fori_loop(0, _NCHUNK // _UNROLL, scan2, 0)
        b2, r2 = _search_4096(h2, h2c, r1)

        p24 = (s1 << 12) | b2

        def scan3(i, _):
            for u in range(_UNROLL):
                off = i * 16 * _UNROLL + u * 16
                key = lax.bitcast_convert_type(
                    rowbuf[pl.ds(off, 16)], jnp.int32)
                pred = (key >> 8) == p24
                plsc.addupdate_scatter(h3, [key & 0xFF], ones, mask=pred)
            return 0

        lax.fori_loop(0, _NCHUNK // _UNROLL, scan3, 0)
        b3, _r3 = _search_256(h3, r2)

        tkey = (s1 << 20) | (b2 << 8) | b3
        tvec = jnp.where(lane == j, tkey, tvec)
        if j + 2 < _ROWS_PER_W:
            copies.append(pltpu.async_copy(
                x_hbm.at[row0 + j + 2], bufs[j % 2], sems[j % 2]))

    tbuf[...] = tvec
    pltpu.sync_copy(tbuf, out_hbm.at[wid])


_sc_thresholds = functools.partial(
    pl.kernel,
    out_type=jax.ShapeDtypeStruct((_NW, 16), jnp.int32),
    mesh=plsc.VectorSubcoreMesh(core_axis_name="c", subcore_axis_name="s"),
    compiler_params=pltpu.CompilerParams(needs_layout_passes=False),
    scratch_types=[
        pltpu.VMEM((_CH,), jnp.float32),
        pltpu.VMEM((_CH,), jnp.float32),
        pltpu.VMEM((_CH,), jnp.float32),
        pltpu.VMEM((4096,), jnp.int32),
        pltpu.VMEM((256,), jnp.int32),
        pltpu.VMEM((4096,), jnp.int32),
        pltpu.VMEM((256,), jnp.int32),
        pltpu.VMEM((256,), jnp.int32),
        pltpu.VMEM((16,), jnp.int32),
        pltpu.SemaphoreType.DMA,
        pltpu.SemaphoreType.DMA,
        pltpu.SemaphoreType.DMA,
    ],
)(_sc_body)


def _mask_body(x_ref, bf_ref, t_ref, o_ref):
    x = x_ref[...]
    bits = lax.bitcast_convert_type(x * bf_ref[...], jnp.int32)
    key = _float_key(bits)
    o_ref[...] = jnp.where(key >= t_ref[...], x, jnp.float32(0.0))


def _apply_mask(inputs, bf, thr):
    rb = 8
    grid = _BATCH // rb
    return pl.pallas_call(
        _mask_body,
        grid=(grid,),
        in_specs=[
            pl.BlockSpec((rb, _CH), lambda i: (i, 0)),
            pl.BlockSpec((1, _CH), lambda i: (0, 0)),
            pl.BlockSpec((rb, 1), lambda i: (i, 0)),
        ],
        out_specs=pl.BlockSpec((rb, _CH), lambda i: (i, 0)),
        out_shape=jax.ShapeDtypeStruct((_BATCH, _CH), jnp.float32),
    )(inputs, bf.reshape(1, _CH), thr)


@jax.jit
def kernel(inputs, dutyCycle):
    bf = _boost_factors(dutyCycle)
    tmat = _sc_thresholds(inputs, bf)
    thr = tmat[:, :_ROWS_PER_W].reshape(_BATCH, 1)
    return _apply_mask(inputs, bf, thr)


# speedup vs baseline: 27.4330x; 2.4600x over previous
import functools

import jax
import jax.numpy as jnp
from jax import lax
from jax.experimental import pallas as pl
from jax.experimental.pallas import tpu as pltpu
from jax.experimental.pallas import tpu_sc as plsc

_BATCH = 128
_CH = 32768
_KWIN = 3276
_BOOST_STRENGTH = 1.0
_TARGET_DUTY = float(_KWIN) / float(_CH)

_NW = 32
_ROWS_PER_W = _BATCH // _NW
_NCHUNK = _CH // 16
_UNROLL = 8


def _bf_body(dc_ref, bf_ref):
    bf_ref[...] = jnp.exp((_TARGET_DUTY - dc_ref[...]) * _BOOST_STRENGTH)


def _boost_factors(dutyCycle):
    dc2 = dutyCycle.reshape(256, 128)
    bf2 = pl.pallas_call(
        _bf_body,
        out_shape=jax.ShapeDtypeStruct((256, 128), jnp.float32),
    )(dc2)
    return bf2.reshape(_CH)


def _float_key(bits):
    return bits ^ ((bits >> 31) & jnp.int32(0x7FFFFFFF))


def _search_chunks(chunks, k_target, reverse=None):
    n = len(chunks)
    rvs = [lax.rev(ch, (0,)) for ch in chunks]
    css = [plsc.cumsum(rv) for rv in rvs]
    tots = [jnp.sum(ch) for ch in chunks]
    above = [None] * n
    acc = jnp.int32(0)
    for c in range(n - 1, -1, -1):
        above[c] = acc
        acc = acc + tots[c]
    n_ge = jnp.zeros((16,), jnp.int32)
    cnt_lt = jnp.zeros((16,), jnp.int32)
    for c in range(n):
        cs_full = css[c] + above[c]
        ge = (cs_full >= k_target).astype(jnp.int32)
        n_ge = n_ge + ge
        cnt_lt = cnt_lt + rvs[c] * (1 - ge)
    return jnp.sum(n_ge) - 1, k_target - jnp.sum(cnt_lt)


def _search_4096(fine_ref, coarse_ref, k_target):
    coarse = [coarse_ref[pl.ds(16 * c, 16)] for c in range(16)]
    g, rg = _search_chunks(coarse, k_target)
    chunk = fine_ref[pl.ds(pl.multiple_of(g * 16, 16), 16)]
    lb, r = _search_chunks([chunk], rg)
    return g * 16 + lb, r


def _search_256(h_ref, k_target):
    chunks = [h_ref[pl.ds(16 * c, 16)] for c in range(16)]
    return _search_chunks(chunks, k_target)


def _sc_body(x_hbm, bf_hbm, out_hbm, buf0, buf1, bfbuf, h1, h1c, h2, h2c,
             h3, tbuf, sem0, sem1, bsem):
    wid = lax.axis_index("s") * 2 + lax.axis_index("c")
    row0 = wid * _ROWS_PER_W
    bufs = (buf0, buf1)
    sems = (sem0, sem1)
    cp_bf = pltpu.async_copy(bf_hbm, bfbuf, bsem)
    copies = [pltpu.async_copy(x_hbm.at[row0], buf0, sem0),
              pltpu.async_copy(x_hbm.at[row0 + 1], buf1, sem1)]
    cp_bf.wait()

    ones = jnp.ones((16,), jnp.int32)
    zeros16 = jnp.zeros((16,), jnp.int32)
    zf16 = jnp.zeros((16,), jnp.float32)
    lane = lax.iota(jnp.int32, 16)
    tvec = zeros16

    for j in range(_ROWS_PER_W):
        rowbuf = bufs[j % 2]
        copies[j].wait()

        @plsc.parallel_loop(0, 256, unroll=_UNROLL)
        def zero12(i):
            h1[pl.ds(i * 16, 16)] = zeros16
            h2[pl.ds(i * 16, 16)] = zeros16
        for c in range(16):
            h1c[pl.ds(16 * c, 16)] = zeros16
            h2c[pl.ds(16 * c, 16)] = zeros16
            h3[pl.ds(16 * c, 16)] = zeros16

        @plsc.parallel_loop(0, _NCHUNK, unroll=_UNROLL)
        def scan1(i):
            off = i * 16
            x = rowbuf[pl.ds(off, 16)]
            b = bfbuf[pl.ds(off, 16)]
            key = _float_key(lax.bitcast_convert_type(x * b, jnp.int32))
            rowbuf[pl.ds(off, 16)] = lax.bitcast_convert_type(
                key, jnp.float32)
            bin1 = (key >> 20) + 2048
            plsc.addupdate_scatter(h1, [bin1], ones)
            plsc.addupdate_scatter(h1c, [bin1 >> 4], ones)
        b1, r1 = _search_4096(h1, h1c, _KWIN)
        s1 = b1 - 2048

        @plsc.parallel_loop(0, _NCHUNK, unroll=_UNROLL)
        def scan2(i):
            off = i * 16
            key = lax.bitcast_convert_type(rowbuf[pl.ds(off, 16)], jnp.int32)
            pred = (key >> 20) == s1
            bin2 = (key >> 8) & 0xFFF
            plsc.addupdate_scatter(h2, [bin2], ones, mask=pred)
            plsc.addupdate_scatter(h2c, [bin2 >> 4], ones, mask=pred)
        b2, r2 = _search_4096(h2, h2c, r1)

        p24 = (s1 << 12) | b2

        @plsc.parallel_loop(0, _NCHUNK, unroll=_UNROLL)
        def scan3(i):
            off = i * 16
            key = lax.bitcast_convert_type(rowbuf[pl.ds(off, 16)], jnp.int32)
            pred = (key >> 8) == p24
            plsc.addupdate_scatter(h3, [key & 0xFF], ones, mask=pred)
        b3, _r3 = _search_256(h3, r2)

        tkey = (s1 << 20) | (b2 << 8) | b3
        tvec = jnp.where(lane == j, tkey, tvec)
        if j + 2 < _ROWS_PER_W:
            copies.append(pltpu.async_copy(
                x_hbm.at[row0 + j + 2], bufs[j % 2], sems[j % 2]))

    tbuf[...] = tvec
    pltpu.sync_copy(tbuf, out_hbm.at[wid])


_sc_thresholds = functools.partial(
    pl.kernel,
    out_type=jax.ShapeDtypeStruct((_NW, 16), jnp.int32),
    mesh=plsc.VectorSubcoreMesh(core_axis_name="c", subcore_axis_name="s"),
    compiler_params=pltpu.CompilerParams(needs_layout_passes=False),
    scratch_types=[
        pltpu.VMEM((_CH,), jnp.float32),
        pltpu.VMEM((_CH,), jnp.float32),
        pltpu.VMEM((_CH,), jnp.float32),
        pltpu.VMEM((4096,), jnp.int32),
        pltpu.VMEM((256,), jnp.int32),
        pltpu.VMEM((4096,), jnp.int32),
        pltpu.VMEM((256,), jnp.int32),
        pltpu.VMEM((256,), jnp.int32),
        pltpu.VMEM((16,), jnp.int32),
        pltpu.SemaphoreType.DMA,
        pltpu.SemaphoreType.DMA,
        pltpu.SemaphoreType.DMA,
    ],
)(_sc_body)


def _mask_body(x_ref, bf_ref, t_ref, o_ref):
    x = x_ref[...]
    bits = lax.bitcast_convert_type(x * bf_ref[...], jnp.int32)
    key = _float_key(bits)
    o_ref[...] = jnp.where(key >= t_ref[...], x, jnp.float32(0.0))


def _apply_mask(inputs, bf, thr):
    rb = 8
    grid = _BATCH // rb
    return pl.pallas_call(
        _mask_body,
        grid=(grid,),
        in_specs=[
            pl.BlockSpec((rb, _CH), lambda i: (i, 0)),
            pl.BlockSpec((1, _CH), lambda i: (0, 0)),
            pl.BlockSpec((rb, 1), lambda i: (i, 0)),
        ],
        out_specs=pl.BlockSpec((rb, _CH), lambda i: (i, 0)),
        out_shape=jax.ShapeDtypeStruct((_BATCH, _CH), jnp.float32),
    )(inputs, bf.reshape(1, _CH), thr)


@jax.jit
def kernel(inputs, dutyCycle):
    bf = _boost_factors(dutyCycle)
    tmat = _sc_thresholds(inputs, bf)
    thr = tmat[:, :_ROWS_PER_W].reshape(_BATCH, 1)
    return _apply_mask(inputs, bf, thr)


# speedup vs baseline: 27.6435x; 1.0077x over previous
import functools

import jax
import jax.numpy as jnp
from jax import lax
from jax.experimental import pallas as pl
from jax.experimental.pallas import tpu as pltpu
from jax.experimental.pallas import tpu_sc as plsc

_BATCH = 128
_CH = 32768
_KWIN = 3276
_BOOST_STRENGTH = 1.0
_TARGET_DUTY = float(_KWIN) / float(_CH)

_NW = 32
_ROWS_PER_W = _BATCH // _NW
_NCHUNK = _CH // 16
_UNROLL = 8


def _bf_body(dc_ref, bf_ref):
    bf_ref[...] = jnp.exp((_TARGET_DUTY - dc_ref[...]) * _BOOST_STRENGTH)


def _boost_factors(dutyCycle):
    dc2 = dutyCycle.reshape(256, 128)
    bf2 = pl.pallas_call(
        _bf_body,
        out_shape=jax.ShapeDtypeStruct((256, 128), jnp.float32),
    )(dc2)
    return bf2.reshape(_CH)


def _float_key(bits):
    return bits ^ ((bits >> 31) & jnp.int32(0x7FFFFFFF))


def _search_chunks(chunks, k_target, reverse=None):
    n = len(chunks)
    rvs = [lax.rev(ch, (0,)) for ch in chunks]
    css = [plsc.cumsum(rv) for rv in rvs]
    tots = [jnp.sum(ch) for ch in chunks]
    above = [None] * n
    acc = jnp.int32(0)
    for c in range(n - 1, -1, -1):
        above[c] = acc
        acc = acc + tots[c]
    n_ge = jnp.zeros((16,), jnp.int32)
    cnt_lt = jnp.zeros((16,), jnp.int32)
    for c in range(n):
        cs_full = css[c] + above[c]
        ge = (cs_full >= k_target).astype(jnp.int32)
        n_ge = n_ge + ge
        cnt_lt = cnt_lt + rvs[c] * (1 - ge)
    return jnp.sum(n_ge) - 1, k_target - jnp.sum(cnt_lt)


def _search_4096(fine_ref, coarse_ref, k_target, lane=None):
    coarse = [coarse_ref[pl.ds(16 * c, 16)] for c in range(16)]
    g, rg = _search_chunks(coarse, k_target)
    chunk = fine_ref[pl.ds(pl.multiple_of(g * 16, 16), 16)]
    lb, r = _search_chunks([chunk], rg)
    if lane is None:
        lane = lax.iota(jnp.int32, 16)
    nb = jnp.sum(jnp.where(lane == lb, chunk, 0))
    return g * 16 + lb, r, nb


def _search_256(h_ref, k_target):
    chunks = [h_ref[pl.ds(16 * c, 16)] for c in range(16)]
    return _search_chunks(chunks, k_target)


def _sc_body(x_hbm, bf_hbm, out_hbm, buf0, buf1, bfbuf, h1, h2, h2c,
             h3, h3c, tbuf, sem0, sem1, bsem):
    wid = lax.axis_index("s") * 2 + lax.axis_index("c")
    row0 = wid * _ROWS_PER_W
    bufs = (buf0, buf1)
    sems = (sem0, sem1)
    cp_bf = pltpu.async_copy(bf_hbm, bfbuf, bsem)
    copies = [pltpu.async_copy(x_hbm.at[row0], buf0, sem0),
              pltpu.async_copy(x_hbm.at[row0 + 1], buf1, sem1)]
    cp_bf.wait()

    ones = jnp.ones((16,), jnp.int32)
    zeros16 = jnp.zeros((16,), jnp.int32)
    lane = lax.iota(jnp.int32, 16)
    tvec = zeros16

    for j in range(_ROWS_PER_W):
        rowbuf = bufs[j % 2]
        copies[j].wait()

        @plsc.parallel_loop(0, 256, unroll=_UNROLL)
        def zero2(i):
            h2[pl.ds(i * 16, 16)] = zeros16
        for c in range(16):
            h1[pl.ds(16 * c, 16)] = zeros16
            h2c[pl.ds(16 * c, 16)] = zeros16

        @plsc.parallel_loop(0, _NCHUNK, unroll=_UNROLL)
        def scan1(i):
            off = i * 16
            x = rowbuf[pl.ds(off, 16)]
            b = bfbuf[pl.ds(off, 16)]
            key = _float_key(lax.bitcast_convert_type(x * b, jnp.int32))
            rowbuf[pl.ds(off, 16)] = lax.bitcast_convert_type(
                key, jnp.float32)
            plsc.addupdate_scatter(h1, [(key >> 24) + 128], ones)
        b1, r1 = _search_256(h1, _KWIN)
        s1 = b1 - 128

        @plsc.parallel_loop(0, _NCHUNK, unroll=_UNROLL)
        def scan2(i):
            off = i * 16
            key = lax.bitcast_convert_type(rowbuf[pl.ds(off, 16)], jnp.int32)
            pred = (key >> 24) == s1
            bin2 = (key >> 12) & 0xFFF
            plsc.addupdate_scatter(h2, [bin2], ones, mask=pred)
            plsc.addupdate_scatter(h2c, [bin2 >> 4], ones, mask=pred)
        b2, r2, nb2 = _search_4096(h2, h2c, r1, lane)
        prefix20 = (s1 << 12) | b2

        def no_scan3():
            return prefix20 << 12

        def do_scan3():
            @plsc.parallel_loop(0, 256, unroll=_UNROLL)
            def zero3(i):
                h3[pl.ds(i * 16, 16)] = zeros16
            for c in range(16):
                h3c[pl.ds(16 * c, 16)] = zeros16

            @plsc.parallel_loop(0, _NCHUNK, unroll=_UNROLL)
            def scan3(i):
                off = i * 16
                key = lax.bitcast_convert_type(
                    rowbuf[pl.ds(off, 16)], jnp.int32)
                pred = (key >> 12) == prefix20
                plsc.addupdate_scatter(h3, [key & 0xFFF], ones, mask=pred)
                plsc.addupdate_scatter(h3c, [(key >> 4) & 0xFF], ones,
                                       mask=pred)
            b3, _r3, _nb3 = _search_4096(h3, h3c, r2, lane)
            return (prefix20 << 12) | b3

        tkey = lax.cond(nb2 == r2, no_scan3, do_scan3)
        tvec = jnp.where(lane == j, tkey, tvec)
        if j + 2 < _ROWS_PER_W:
            copies.append(pltpu.async_copy(
                x_hbm.at[row0 + j + 2], bufs[j % 2], sems[j % 2]))

    tbuf[...] = tvec
    pltpu.sync_copy(tbuf, out_hbm.at[wid])


_sc_thresholds = functools.partial(
    pl.kernel,
    out_type=jax.ShapeDtypeStruct((_NW, 16), jnp.int32),
    mesh=plsc.VectorSubcoreMesh(core_axis_name="c", subcore_axis_name="s"),
    compiler_params=pltpu.CompilerParams(needs_layout_passes=False),
    scratch_types=[
        pltpu.VMEM((_CH,), jnp.float32),
        pltpu.VMEM((_CH,), jnp.float32),
        pltpu.VMEM((_CH,), jnp.float32),
        pltpu.VMEM((256,), jnp.int32),
        pltpu.VMEM((4096,), jnp.int32),
        pltpu.VMEM((256,), jnp.int32),
        pltpu.VMEM((4096,), jnp.int32),
        pltpu.VMEM((256,), jnp.int32),
        pltpu.VMEM((16,), jnp.int32),
        pltpu.SemaphoreType.DMA,
        pltpu.SemaphoreType.DMA,
        pltpu.SemaphoreType.DMA,
    ],
)(_sc_body)


def _mask_body(x_ref, bf_ref, t_ref, o_ref):
    x = x_ref[...]
    bits = lax.bitcast_convert_type(x * bf_ref[...], jnp.int32)
    key = _float_key(bits)
    o_ref[...] = jnp.where(key >= t_ref[...], x, jnp.float32(0.0))


def _apply_mask(inputs, bf, thr):
    rb = 8
    grid = _BATCH // rb
    return pl.pallas_call(
        _mask_body,
        grid=(grid,),
        in_specs=[
            pl.BlockSpec((rb, _CH), lambda i: (i, 0)),
            pl.BlockSpec((1, _CH), lambda i: (0, 0)),
            pl.BlockSpec((rb, 1), lambda i: (i, 0)),
        ],
        out_specs=pl.BlockSpec((rb, _CH), lambda i: (i, 0)),
        out_shape=jax.ShapeDtypeStruct((_BATCH, _CH), jnp.float32),
    )(inputs, bf.reshape(1, _CH), thr)


@jax.jit
def kernel(inputs, dutyCycle):
    bf = _boost_factors(dutyCycle)
    tmat = _sc_thresholds(inputs, bf)
    thr = tmat[:, :_ROWS_PER_W].reshape(_BATCH, 1)
    return _apply_mask(inputs, bf, thr)
